# baseline (speedup 1.0000x reference)
import functools

import jax
import jax.numpy as jnp
from jax import lax
from jax.experimental import pallas as pl
from jax.experimental.pallas import tpu as pltpu
from jax.experimental.pallas import tpu_sc as plsc

N = 10000
E = 320000
IN_CH = 128
OUT_CH = 128
H = 8
B = 4
A = 3
DH = 16
F = B * DH
FAN = H * B * A

BLK = 1000
NBLK = N // BLK

CHUNK = 4000
NCH = E // CHUNK
VPC = CHUNK // 16
EPT = E // 32
CCH = 2000
NEG = -3.4e38


def _prep_body(wf_ref, x_ref, o_ref):
    o_ref[...] = lax.dot_general(
        wf_ref[...], x_ref[...], (((0,), (1,)), ((), ())),
        preferred_element_type=jnp.float32)


def _prep(W_flat, x):
    return pl.pallas_call(
        _prep_body,
        out_shape=jax.ShapeDtypeStruct((F, N), jnp.float32),
    )(W_flat, x)


def _agg_body(flatT, srcv, dstv, outA, outM, outC,
              row0, row1, aA0, aA1, aM0, aM1, cntb, tmpi, tmpf, scr,
              spillS, spillD, scr2, spill2S, spill2D,
              sAs, sAd, sBs, sBd, semA, semB):
    wid = lax.axis_index("s") * 2 + lax.axis_index("c")
    f0 = wid * 2
    pltpu.sync_copy(flatT.at[f0], row0)
    pltpu.sync_copy(flatT.at[f0 + 1], row1)

    z16 = jnp.zeros((16,), jnp.float32)
    neg16 = jnp.full((16,), NEG, jnp.float32)

    def zbody(i, c):
        s = pl.ds(i * 16, 16)
        aA0[s] = z16
        aA1[s] = z16
        aM0[s] = neg16
        aM1[s] = neg16
        cntb[s] = z16
        return c
    lax.fori_loop(0, (N + 16) // 16, zbody, 0)

    iota16 = lax.iota(jnp.int32, 16)
    ones16 = jnp.ones((16,), jnp.bool_)
    onesf = jnp.ones((16,), jnp.float32)
    shifts = (1, 2, 4, 8)
    cdown = [jnp.maximum(iota16 - sh, 0) for sh in shifts]
    cge = [iota16 >= sh for sh in shifts]
    cup = jnp.minimum(iota16 + 1, 15)
    is15 = iota16 == 15

    def start(c, bs, bd, sem):
        pltpu.async_copy(srcv.at[pl.ds(c * CHUNK, CHUNK)], bs, sem)
        pltpu.async_copy(dstv.at[pl.ds(c * CHUNK, CHUNK)], bd, sem)

    def wait(c, bs, bd, sem):
        pltpu.make_async_copy(srcv.at[pl.ds(c * CHUNK, CHUNK)], bs, sem).wait()
        pltpu.make_async_copy(dstv.at[pl.ds(c * CHUNK, CHUNK)], bd, sem).wait()

    def group(bs, bd, base, ti, tf):
        src16 = bs[pl.ds(base, 16)]
        dst16 = bd[pl.ds(base, 16)]
        dst_s, order = plsc.sort_key_val(dst16, iota16)
        ti[...] = src16
        src_s = plsc.load_gather(ti, [order])
        ti[...] = dst_s
        masks = []
        for k in range(4):
            d_sh = plsc.load_gather(ti, [cdown[k]])
            masks.append((dst_s == d_sh) & cge[k])
        d_n = plsc.load_gather(ti, [cup])
        m_last = (dst_s != d_n) | is15
        idx_last = jnp.where(m_last, dst_s, N)
        for row, aA, aM in ((row0, aA0, aM0), (row1, aA1, aM1)):
            msg = plsc.load_gather(row, [src_s])
            w = msg
            u = msg
            for k in range(4):
                tf[...] = w
                t = plsc.load_gather(tf, [cdown[k]])
                w = jnp.where(masks[k], jnp.maximum(w, t), w)
                tf[...] = u
                t2 = plsc.load_gather(tf, [cdown[k]])
                u = u + jnp.where(masks[k], t2, 0.0)
            aa = plsc.load_gather(aA, [idx_last])
            plsc.store_scatter(aA, [idx_last], aa + u)
            old = plsc.load_gather(aM, [idx_last])
            plsc.store_scatter(aM, [idx_last], jnp.maximum(old, w))

    def phase1(bs, bd, base, sc, spS, spD, ptr):
        s = pl.ds(base, 16)
        src16 = bs[s]
        dst16 = bd[s]
        plsc.store_scatter(sc, [dst16], iota16)
        back = plsc.load_gather(sc, [dst16])
        win = back == iota16
        idx_w = jnp.where(win, dst16, N)
        for row, aA, aM in ((row0, aA0, aM0), (row1, aA1, aM1)):
            msg = plsc.load_gather(row, [src16])
            aa = plsc.load_gather(aA, [idx_w])
            plsc.store_scatter(aA, [idx_w], aa + msg)
            old = plsc.load_gather(aM, [idx_w])
            plsc.store_scatter(aM, [idx_w], jnp.maximum(old, msg))
        spill = jnp.logical_not(win)
        plsc.store_compressed(spS.at[pl.ds(ptr, 16)], src16, mask=spill)
        plsc.store_compressed(spD.at[pl.ds(ptr, 16)], dst16, mask=spill)
        return ptr + jnp.sum(jnp.where(spill, 1, 0))

    def process(bs, bd, c):
        def vbody(v, ptrs):
            pa = phase1(bs, bd, v * 32, scr, spillS, spillD, ptrs[0])
            pb = phase1(bs, bd, v * 32 + 16, scr2, spill2S, spill2D, ptrs[1])
            return (pa, pb)
        ptrA, ptrB = lax.fori_loop(0, VPC // 2, vbody, (0, 0))
        for spS, spD, ptr in ((spillS, spillD, ptrA),
                              (spill2S, spill2D, ptrB)):
            spS[pl.ds(ptr, 16)] = jnp.zeros((16,), jnp.int32)
            spD[pl.ds(ptr, 16)] = jnp.full((16,), N, jnp.int32)

            def sbody(g, carry):
                group(spS, spD, g * 16, tmpi, tmpf)
                return carry
            lax.fori_loop(0, (ptr + 15) // 16, sbody, 0)

    start(0, sAs, sAd, semA)

    def pbody(p, carry):
        c0 = 2 * p
        c1 = c0 + 1
        start(c1, sBs, sBd, semB)
        wait(c0, sAs, sAd, semA)
        process(sAs, sAd, c0)

        @pl.when(p < NCH // 2 - 1)
        def _():
            start(c0 + 2, sAs, sAd, semA)

        wait(c1, sBs, sBd, semB)
        process(sBs, sBd, c1)
        return carry
    lax.fori_loop(0, NCH // 2, pbody, 0)

    def cbody(i, carry):
        pltpu.sync_copy(dstv.at[pl.ds(wid * EPT + i * CCH, CCH)],
                        sAd.at[pl.ds(0, CCH)])

        def cvbody(v, c2):
            dst16 = sAd[pl.ds(v * 16, 16)]
            dst_s, _ = plsc.sort_key_val(dst16, iota16)
            tmpi[...] = dst_s
            masks = []
            for k in range(4):
                d_sh = plsc.load_gather(tmpi, [cdown[k]])
                masks.append((dst_s == d_sh) & cge[k])
            d_n = plsc.load_gather(tmpi, [cup])
            m_last = (dst_s != d_n) | is15
            idx_last = jnp.where(m_last, dst_s, N)
            u = onesf
            for k in range(4):
                tmpf[...] = u
                t2 = plsc.load_gather(tmpf, [cdown[k]])
                u = u + jnp.where(masks[k], t2, 0.0)
            cc = plsc.load_gather(cntb, [idx_last])
            plsc.store_scatter(cntb, [idx_last], cc + u)
            return c2
        lax.fori_loop(0, CCH // 16, cvbody, 0)
        return carry
    lax.fori_loop(0, EPT // CCH, cbody, 0)

    pltpu.sync_copy(aA0.at[pl.ds(0, N)], outA.at[f0])
    pltpu.sync_copy(aA1.at[pl.ds(0, N)], outA.at[f0 + 1])
    pltpu.sync_copy(aM0.at[pl.ds(0, N)], outM.at[f0])
    pltpu.sync_copy(aM1.at[pl.ds(0, N)], outM.at[f0 + 1])
    pltpu.sync_copy(cntb.at[pl.ds(0, N)], outC.at[wid])


def _make_agg():
    mesh = plsc.VectorSubcoreMesh(core_axis_name="c", subcore_axis_name="s")
    return pl.kernel(
        _agg_body,
        mesh=mesh,
        compiler_params=pltpu.CompilerParams(needs_layout_passes=False,
                                             use_tc_tiling_on_sc=False),
        out_type=(
            jax.ShapeDtypeStruct((F, N), jnp.float32),
            jax.ShapeDtypeStruct((F, N), jnp.float32),
            jax.ShapeDtypeStruct((32, N), jnp.float32),
        ),
        scratch_types=[
            pltpu.VMEM((N,), jnp.float32),
            pltpu.VMEM((N,), jnp.float32),
            pltpu.VMEM((N + 16,), jnp.float32),
            pltpu.VMEM((N + 16,), jnp.float32),
            pltpu.VMEM((N + 16,), jnp.float32),
            pltpu.VMEM((N + 16,), jnp.float32),
            pltpu.VMEM((N + 16,), jnp.float32),
            pltpu.VMEM((16,), jnp.int32),
            pltpu.VMEM((16,), jnp.float32),
            pltpu.VMEM((N,), jnp.int32),
            pltpu.VMEM((CHUNK + 16,), jnp.int32),
            pltpu.VMEM((CHUNK + 16,), jnp.int32),
            pltpu.VMEM((N,), jnp.int32),
            pltpu.VMEM((CHUNK + 16,), jnp.int32),
            pltpu.VMEM((CHUNK + 16,), jnp.int32),
            pltpu.VMEM((CHUNK,), jnp.int32),
            pltpu.VMEM((CHUNK,), jnp.int32),
            pltpu.VMEM((CHUNK,), jnp.int32),
            pltpu.VMEM((CHUNK,), jnp.int32),
            pltpu.SemaphoreType.DMA,
            pltpu.SemaphoreType.DMA,
        ],
    )


def _tx_body(addT_ref, maxT_ref, cntP_ref, add_ref, mx_ref, cnt_ref):
    r = lax.broadcasted_iota(jnp.int32, (F, F), 0)
    c = lax.broadcasted_iota(jnp.int32, (F, F), 1)
    eye = (r == c).astype(jnp.float32)
    dn = (((0,), (0,)), ((), ()))
    add_ref[...] = lax.dot_general(addT_ref[...], eye, dn,
                                   preferred_element_type=jnp.float32)
    mx_ref[...] = lax.dot_general(maxT_ref[...], eye, dn,
                                  preferred_element_type=jnp.float32)
    cnt_ref[...] = lax.dot_general(cntP_ref[...],
                                   jnp.ones((32, 1), jnp.float32), dn,
                                   preferred_element_type=jnp.float32)


def _tx(addT, maxT, cntP):
    return pl.pallas_call(
        _tx_body,
        out_shape=(
            jax.ShapeDtypeStruct((N, F), jnp.float32),
            jax.ShapeDtypeStruct((N, F), jnp.float32),
            jax.ShapeDtypeStruct((N, 1), jnp.float32),
        ),
    )(addT, maxT, cntP)


def _combine_body(x_ref, add_ref, max_ref, cnt_ref, cwT_ref, cb_ref,
                  bias_ref, o_ref):
    x = x_ref[...]
    logits = jnp.dot(x, cwT_ref[...], preferred_element_type=jnp.float32)
    logits = logits + cb_ref[...]
    m = jnp.max(logits, axis=1, keepdims=True)
    s = jnp.exp(logits - m)
    cnt = cnt_ref[...]
    add = add_ref[...]
    mean = add / jnp.maximum(cnt, 1.0)
    mx = jnp.where(cnt > 0.0, max_ref[...], 0.0)
    aggs = (add, mean, mx)
    out = []
    for h in range(H):
        num = jnp.zeros((BLK, DH), jnp.float32)
        den = jnp.zeros((BLK, 1), jnp.float32)
        for b in range(B):
            for a in range(A):
                w = s[:, h * (B * A) + b * A + a][:, None]
                num = num + w * aggs[a][:, b * DH:(b + 1) * DH]
                den = den + w
        out.append(num / den)
    o_ref[...] = jnp.concatenate(out, axis=1) + bias_ref[...]


def _combine(x, add, mx, cnt, cwT, cb, bias):
    return pl.pallas_call(
        _combine_body,
        grid=(NBLK,),
        in_specs=[
            pl.BlockSpec((BLK, IN_CH), lambda i: (i, 0)),
            pl.BlockSpec((BLK, F), lambda i: (i, 0)),
            pl.BlockSpec((BLK, F), lambda i: (i, 0)),
            pl.BlockSpec((BLK, 1), lambda i: (i, 0)),
            pl.BlockSpec((IN_CH, FAN), lambda i: (0, 0)),
            pl.BlockSpec((1, FAN), lambda i: (0, 0)),
            pl.BlockSpec((1, OUT_CH), lambda i: (0, 0)),
        ],
        out_specs=pl.BlockSpec((BLK, OUT_CH), lambda i: (i, 0)),
        out_shape=jax.ShapeDtypeStruct((N, OUT_CH), jnp.float32),
    )(x, add, mx, cnt, cwT, cb, bias)


def kernel(x, edge_index, W_bases, comb_W, comb_b, bias):
    ei = edge_index.astype(jnp.int32)
    src = ei[0]
    dst = ei[1]
    W_flat = jnp.transpose(W_bases, (1, 0, 2)).reshape(IN_CH, F)
    flatT = _prep(W_flat, x)
    aggAT, aggMT, cntP = _make_agg()(flatT, src, dst)
    add, mx, cnt = _tx(aggAT, aggMT, cntP)
    return _combine(x, add, mx, cnt, comb_W.T,
                    comb_b[None, :], bias[None, :])


# speedup vs baseline: 1.2597x; 1.2597x over previous
import functools

import jax
import jax.numpy as jnp
from jax import lax
from jax.experimental import pallas as pl
from jax.experimental.pallas import tpu as pltpu
from jax.experimental.pallas import tpu_sc as plsc

N = 10000
E = 320000
IN_CH = 128
OUT_CH = 128
H = 8
B = 4
A = 3
DH = 16
F = B * DH
FAN = H * B * A

BLK = 1000
NBLK = N // BLK

CHUNK = 4000
NCH = E // CHUNK
VPC = CHUNK // 16
EPT = E // 32
CCH = 2000
NEG = -3.4e38


def _prep_body(wf_ref, x_ref, o_ref):
    o_ref[...] = lax.dot_general(
        wf_ref[...], x_ref[...], (((0,), (1,)), ((), ())),
        preferred_element_type=jnp.float32)


def _prep(W_flat, x):
    return pl.pallas_call(
        _prep_body,
        out_shape=jax.ShapeDtypeStruct((F, N), jnp.float32),
    )(W_flat, x)


def _agg_body(flatT, srcv, dstv, outA, outM, outC,
              row0, row1, aA0, aA1, aM0, aM1, cntb, tmpi, tmpf, scr,
              spillS, spillD, scr2, spill2S, spill2D,
              sAs, sAd, sBs, sBd, semA, semB):
    wid = lax.axis_index("s") * 2 + lax.axis_index("c")
    f0 = wid * 2
    pltpu.sync_copy(flatT.at[f0], row0)
    pltpu.sync_copy(flatT.at[f0 + 1], row1)

    z16 = jnp.zeros((16,), jnp.float32)
    neg16 = jnp.full((16,), NEG, jnp.float32)

    def zbody(i, c):
        s = pl.ds(i * 16, 16)
        aA0[s] = z16
        aA1[s] = z16
        aM0[s] = neg16
        aM1[s] = neg16
        cntb[s] = z16
        return c
    lax.fori_loop(0, (N + 16) // 16, zbody, 0)

    iota16 = lax.iota(jnp.int32, 16)
    ones16 = jnp.ones((16,), jnp.bool_)
    onesf = jnp.ones((16,), jnp.float32)
    shifts = (1, 2, 4, 8)
    cdown = [jnp.maximum(iota16 - sh, 0) for sh in shifts]
    cge = [iota16 >= sh for sh in shifts]
    cup = jnp.minimum(iota16 + 1, 15)
    is15 = iota16 == 15

    def start(c, bs, bd, sem):
        pltpu.async_copy(srcv.at[pl.ds(c * CHUNK, CHUNK)], bs, sem)
        pltpu.async_copy(dstv.at[pl.ds(c * CHUNK, CHUNK)], bd, sem)

    def wait(c, bs, bd, sem):
        pltpu.make_async_copy(srcv.at[pl.ds(c * CHUNK, CHUNK)], bs, sem).wait()
        pltpu.make_async_copy(dstv.at[pl.ds(c * CHUNK, CHUNK)], bd, sem).wait()

    def group(bs, bd, base, ti, tf):
        src16 = bs[pl.ds(base, 16)]
        dst16 = bd[pl.ds(base, 16)]
        dst_s, order = plsc.sort_key_val(dst16, iota16)
        ti[...] = src16
        src_s = plsc.load_gather(ti, [order])
        ti[...] = dst_s
        masks = []
        for k in range(4):
            d_sh = plsc.load_gather(ti, [cdown[k]])
            masks.append((dst_s == d_sh) & cge[k])
        d_n = plsc.load_gather(ti, [cup])
        m_last = (dst_s != d_n) | is15
        idx_last = jnp.where(m_last, dst_s, N)
        for row, aA, aM in ((row0, aA0, aM0), (row1, aA1, aM1)):
            msg = plsc.load_gather(row, [src_s])
            w = msg
            u = msg
            for k in range(4):
                tf[...] = w
                t = plsc.load_gather(tf, [cdown[k]])
                w = jnp.where(masks[k], jnp.maximum(w, t), w)
                tf[...] = u
                t2 = plsc.load_gather(tf, [cdown[k]])
                u = u + jnp.where(masks[k], t2, 0.0)
            aa = plsc.load_gather(aA, [idx_last])
            plsc.store_scatter(aA, [idx_last], aa + u)
            old = plsc.load_gather(aM, [idx_last])
            plsc.store_scatter(aM, [idx_last], jnp.maximum(old, w))

    def phase1(bs, bd, base, sc, spS, spD, ptr):
        s = pl.ds(base, 16)
        src16 = bs[s]
        dst16 = bd[s]
        plsc.store_scatter(sc, [dst16], iota16)
        back = plsc.load_gather(sc, [dst16])
        win = back == iota16
        idx_w = jnp.where(win, dst16, N)
        for row, aA, aM in ((row0, aA0, aM0), (row1, aA1, aM1)):
            msg = plsc.load_gather(row, [src16])
            aa = plsc.load_gather(aA, [idx_w])
            plsc.store_scatter(aA, [idx_w], aa + msg)
            old = plsc.load_gather(aM, [idx_w])
            plsc.store_scatter(aM, [idx_w], jnp.maximum(old, msg))
        spill = jnp.logical_not(win)
        plsc.store_compressed(spS.at[pl.ds(ptr, 16)], src16, mask=spill)
        plsc.store_compressed(spD.at[pl.ds(ptr, 16)], dst16, mask=spill)
        return ptr + jnp.sum(jnp.where(spill, 1, 0))

    def process(bs, bd, c):
        def vbody(v, ptrs):
            pa = phase1(bs, bd, v * 32, scr, spillS, spillD, ptrs[0])
            pb = phase1(bs, bd, v * 32 + 16, scr2, spill2S, spill2D, ptrs[1])
            return (pa, pb)
        ptrA, ptrB = lax.fori_loop(0, VPC // 2, vbody, (0, 0))
        for spS, spD, ptr in ((spillS, spillD, ptrA),
                              (spill2S, spill2D, ptrB)):
            spS[pl.ds(ptr, 16)] = jnp.zeros((16,), jnp.int32)
            spD[pl.ds(ptr, 16)] = jnp.full((16,), N, jnp.int32)

            def sbody(g, carry):
                group(spS, spD, g * 16, tmpi, tmpf)
                return carry
            lax.fori_loop(0, (ptr + 15) // 16, sbody, 0)

    start(0, sAs, sAd, semA)

    def pbody(p, carry):
        c0 = 2 * p
        c1 = c0 + 1
        start(c1, sBs, sBd, semB)
        wait(c0, sAs, sAd, semA)
        process(sAs, sAd, c0)

        @pl.when(p < NCH // 2 - 1)
        def _():
            start(c0 + 2, sAs, sAd, semA)

        wait(c1, sBs, sBd, semB)
        process(sBs, sBd, c1)
        return carry
    lax.fori_loop(0, NCH // 2, pbody, 0)

    def cbody(i, carry):
        pltpu.sync_copy(dstv.at[pl.ds(wid * EPT + i * CCH, CCH)],
                        sAd.at[pl.ds(0, CCH)])

        def cvbody(v, c2):
            dst16 = sAd[pl.ds(v * 16, 16)]
            dst_s, _ = plsc.sort_key_val(dst16, iota16)
            tmpi[...] = dst_s
            masks = []
            for k in range(4):
                d_sh = plsc.load_gather(tmpi, [cdown[k]])
                masks.append((dst_s == d_sh) & cge[k])
            d_n = plsc.load_gather(tmpi, [cup])
            m_last = (dst_s != d_n) | is15
            idx_last = jnp.where(m_last, dst_s, N)
            u = onesf
            for k in range(4):
                tmpf[...] = u
                t2 = plsc.load_gather(tmpf, [cdown[k]])
                u = u + jnp.where(masks[k], t2, 0.0)
            cc = plsc.load_gather(cntb, [idx_last])
            plsc.store_scatter(cntb, [idx_last], cc + u)
            return c2
        lax.fori_loop(0, CCH // 16, cvbody, 0)
        return carry
    lax.fori_loop(0, EPT // CCH, cbody, 0)

    pltpu.sync_copy(aA0.at[pl.ds(0, N)], outA.at[f0])
    pltpu.sync_copy(aA1.at[pl.ds(0, N)], outA.at[f0 + 1])
    pltpu.sync_copy(aM0.at[pl.ds(0, N)], outM.at[f0])
    pltpu.sync_copy(aM1.at[pl.ds(0, N)], outM.at[f0 + 1])
    pltpu.sync_copy(cntb.at[pl.ds(0, N)], outC.at[wid])


def _make_agg():
    mesh = plsc.VectorSubcoreMesh(core_axis_name="c", subcore_axis_name="s")
    return pl.kernel(
        _agg_body,
        mesh=mesh,
        compiler_params=pltpu.CompilerParams(needs_layout_passes=False,
                                             use_tc_tiling_on_sc=False),
        out_type=(
            jax.ShapeDtypeStruct((F, N), jnp.float32),
            jax.ShapeDtypeStruct((F, N), jnp.float32),
            jax.ShapeDtypeStruct((32, N), jnp.float32),
        ),
        scratch_types=[
            pltpu.VMEM((N,), jnp.float32),
            pltpu.VMEM((N,), jnp.float32),
            pltpu.VMEM((N + 16,), jnp.float32),
            pltpu.VMEM((N + 16,), jnp.float32),
            pltpu.VMEM((N + 16,), jnp.float32),
            pltpu.VMEM((N + 16,), jnp.float32),
            pltpu.VMEM((N + 16,), jnp.float32),
            pltpu.VMEM((16,), jnp.int32),
            pltpu.VMEM((16,), jnp.float32),
            pltpu.VMEM((N,), jnp.int32),
            pltpu.VMEM((CHUNK + 16,), jnp.int32),
            pltpu.VMEM((CHUNK + 16,), jnp.int32),
            pltpu.VMEM((N,), jnp.int32),
            pltpu.VMEM((CHUNK + 16,), jnp.int32),
            pltpu.VMEM((CHUNK + 16,), jnp.int32),
            pltpu.VMEM((CHUNK,), jnp.int32),
            pltpu.VMEM((CHUNK,), jnp.int32),
            pltpu.VMEM((CHUNK,), jnp.int32),
            pltpu.VMEM((CHUNK,), jnp.int32),
            pltpu.SemaphoreType.DMA,
            pltpu.SemaphoreType.DMA,
        ],
    )


def _tx_body(addT_ref, maxT_ref, cntP_ref, add_ref, mx_ref, cnt_ref):
    r = lax.broadcasted_iota(jnp.int32, (F, F), 0)
    c = lax.broadcasted_iota(jnp.int32, (F, F), 1)
    eye = (r == c).astype(jnp.float32)
    dn = (((0,), (0,)), ((), ()))
    add_ref[...] = lax.dot_general(addT_ref[...], eye, dn,
                                   preferred_element_type=jnp.float32)
    mx_ref[...] = lax.dot_general(maxT_ref[...], eye, dn,
                                  preferred_element_type=jnp.float32)
    cnt_ref[...] = lax.dot_general(cntP_ref[...],
                                   jnp.ones((32, 1), jnp.float32), dn,
                                   preferred_element_type=jnp.float32)


def _tx(addT, maxT, cntP):
    return pl.pallas_call(
        _tx_body,
        out_shape=(
            jax.ShapeDtypeStruct((N, F), jnp.float32),
            jax.ShapeDtypeStruct((N, F), jnp.float32),
            jax.ShapeDtypeStruct((N, 1), jnp.float32),
        ),
    )(addT, maxT, cntP)


def _combine_body(x_ref, add_ref, max_ref, cnt_ref, cwT_ref, cb_ref,
                  bias_ref, o_ref):
    x = x_ref[...]
    logits = jnp.dot(x, cwT_ref[...], preferred_element_type=jnp.float32)
    logits = logits + cb_ref[...]
    m = jnp.max(logits, axis=1, keepdims=True)
    s = jnp.exp(logits - m)
    cnt = cnt_ref[...]
    add = add_ref[...]
    mean = add / jnp.maximum(cnt, 1.0)
    mx = jnp.where(cnt > 0.0, max_ref[...], 0.0)
    aggs = (add, mean, mx)

    def split(v):
        hi = v.astype(jnp.bfloat16)
        lo = (v - hi.astype(jnp.float32)).astype(jnp.bfloat16)
        return hi, lo

    def mm(hilo, m):
        hi, lo = hilo
        mb = m.astype(jnp.bfloat16)
        return (jnp.dot(hi, mb, preferred_element_type=jnp.float32)
                + jnp.dot(lo, mb, preferred_element_type=jnp.float32))

    jj = lax.broadcasted_iota(jnp.int32, (FAN, OUT_CH), 0)
    hd = lax.broadcasted_iota(jnp.int32, (FAN, OUT_CH), 1)
    fd = lax.broadcasted_iota(jnp.int32, (F, OUT_CH), 0)
    hd2 = lax.broadcasted_iota(jnp.int32, (F, OUT_CH), 1)
    sS = split(s)
    aS = [split(a) for a in aggs]
    den = mm(sS, ((jj // (B * A)) == (hd // DH)).astype(jnp.float32))
    z = jnp.zeros((BLK, OUT_CH), jnp.float32)
    for b in range(B):
        rb = ((fd == b * DH + hd2 % DH)).astype(jnp.float32)
        for a in range(A):
            pba = (jj == (hd // DH) * (B * A) + b * A + a)
            z = z + mm(sS, pba.astype(jnp.float32)) * mm(aS[a], rb)
    o_ref[...] = z / den + bias_ref[...]


def _combine(x, add, mx, cnt, cwT, cb, bias):
    return pl.pallas_call(
        _combine_body,
        grid=(NBLK,),
        in_specs=[
            pl.BlockSpec((BLK, IN_CH), lambda i: (i, 0)),
            pl.BlockSpec((BLK, F), lambda i: (i, 0)),
            pl.BlockSpec((BLK, F), lambda i: (i, 0)),
            pl.BlockSpec((BLK, 1), lambda i: (i, 0)),
            pl.BlockSpec((IN_CH, FAN), lambda i: (0, 0)),
            pl.BlockSpec((1, FAN), lambda i: (0, 0)),
            pl.BlockSpec((1, OUT_CH), lambda i: (0, 0)),
        ],
        out_specs=pl.BlockSpec((BLK, OUT_CH), lambda i: (i, 0)),
        out_shape=jax.ShapeDtypeStruct((N, OUT_CH), jnp.float32),
    )(x, add, mx, cnt, cwT, cb, bias)


def kernel(x, edge_index, W_bases, comb_W, comb_b, bias):
    ei = edge_index.astype(jnp.int32)
    src = ei[0]
    dst = ei[1]
    W_flat = jnp.transpose(W_bases, (1, 0, 2)).reshape(IN_CH, F)
    flatT = _prep(W_flat, x)
    aggAT, aggMT, cntP = _make_agg()(flatT, src, dst)
    add, mx, cnt = _tx(aggAT, aggMT, cntP)
    return _combine(x, add, mx, cnt, comb_W.T,
                    comb_b[None, :], bias[None, :])
